# Initial kernel scaffold; baseline (speedup 1.0000x reference)
import jax
import jax.numpy as jnp
from jax.experimental import pallas as pl


def kernel(x_BSD, Wr, br, W1, b1, W2, b2, biases_N):
    raise NotImplementedError("write your pallas kernel here")


# speedup vs baseline: 2.1238x; 2.1238x over previous
import functools

import jax
import jax.numpy as jnp
from jax.experimental import pallas as pl
from jax.experimental.pallas import tpu as pltpu


def _moe_dense_body(x_ref, wr_ref, gate_b_ref, sel_b_ref, w1_ref, w2_ref,
                    out_ref, i0_s, i1_s, v0_s, v1_s, *, n_experts):
    e = pl.program_id(1)

    @pl.when(e == 0)
    def _router():
        x = x_ref[...]
        s = jax.lax.dot_general(x, wr_ref[...], (((1,), (0,)), ((), ())),
                                preferred_element_type=jnp.float32)
        s = s + gate_b_ref[...]
        sb = s + sel_b_ref[...]
        lane = jax.lax.broadcasted_iota(jnp.int32, sb.shape, 1)
        i0 = jnp.argmax(sb, axis=1)[:, None]
        sb2 = jnp.where(lane == i0, -jnp.inf, sb)
        i1 = jnp.argmax(sb2, axis=1)[:, None]
        s0 = jnp.sum(jnp.where(lane == i0, s, 0.0), axis=1, keepdims=True)
        s1 = jnp.sum(jnp.where(lane == i1, s, 0.0), axis=1, keepdims=True)
        i0_s[...] = i0
        i1_s[...] = i1
        v0_s[...] = jax.nn.sigmoid(s0 - s1)
        v1_s[...] = jax.nn.sigmoid(s1 - s0)

    xb = x_ref[...].astype(jnp.bfloat16)
    h = jax.lax.dot_general(xb, w1_ref[0], (((1,), (0,)), ((), ())),
                            preferred_element_type=jnp.float32)
    H = w2_ref.shape[1]
    a = h[:, :H]
    b = h[:, H:]
    g = (a * jax.nn.sigmoid(a) * b).astype(jnp.bfloat16)
    o = jax.lax.dot_general(g, w2_ref[0], (((1,), (0,)), ((), ())),
                            preferred_element_type=jnp.float32)
    w_e = (jnp.where(i0_s[...] == e, v0_s[...], 0.0)
           + jnp.where(i1_s[...] == e, v1_s[...], 0.0))
    contrib = w_e * o

    @pl.when(e == 0)
    def _init():
        out_ref[...] = contrib

    @pl.when(e != 0)
    def _acc():
        out_ref[...] = out_ref[...] + contrib


def kernel(x_BSD, Wr, br, W1, b1, W2, b2, biases_N):
    B, S, D = x_BSD.shape
    N = Wr.shape[1]
    H = W2.shape[1]
    M = B * S
    BT = min(2048, M)
    num_t = M // BT

    x_MD = x_BSD.reshape(M, D)
    gate_b = br.reshape(1, N)
    sel_b = biases_N.reshape(1, N)

    W1b = W1.astype(jnp.bfloat16)
    W2b = W2.astype(jnp.bfloat16)

    grid = (num_t, N)
    out = pl.pallas_call(
        functools.partial(_moe_dense_body, n_experts=N),
        grid=grid,
        in_specs=[
            pl.BlockSpec((BT, D), lambda t, e: (t, 0)),
            pl.BlockSpec((D, N), lambda t, e: (0, 0)),
            pl.BlockSpec((1, N), lambda t, e: (0, 0)),
            pl.BlockSpec((1, N), lambda t, e: (0, 0)),
            pl.BlockSpec((1, D, 2 * H), lambda t, e: (e, 0, 0)),
            pl.BlockSpec((1, H, D), lambda t, e: (e, 0, 0)),
        ],
        out_specs=pl.BlockSpec((BT, D), lambda t, e: (t, 0)),
        out_shape=jax.ShapeDtypeStruct((M, D), jnp.float32),
        scratch_shapes=[
            pltpu.VMEM((BT, 1), jnp.int32),
            pltpu.VMEM((BT, 1), jnp.int32),
            pltpu.VMEM((BT, 1), jnp.float32),
            pltpu.VMEM((BT, 1), jnp.float32),
        ],
    )(x_MD, Wr, gate_b, sel_b, W1b, W2b)
    return out.reshape(B, S, D)


# speedup vs baseline: 2.1919x; 1.0321x over previous
import functools

import jax
import jax.numpy as jnp
from jax import lax
from jax.experimental import pallas as pl
from jax.experimental.pallas import tpu as pltpu
from jax.experimental.pallas import tpu_sc as plsc

M = 8192
D = 768
H2 = 1536
H = 768
N = 8
K = 2
P = M * K
BT = 512
NW = 32
CH = P // NW
P_PAD = P + N * BT
NT = P_PAD // BT
SUB = 64
NSUB = CH // SUB


def _router_body(x_ref, wr_ref, gate_b_ref, sel_b_ref,
                 eidx_ref, gates_ref, cnts_ref):
    x = x_ref[...]
    s = lax.dot_general(x, wr_ref[...], (((1,), (0,)), ((), ())),
                        preferred_element_type=jnp.float32)
    s = s + gate_b_ref[...]
    sb = s + sel_b_ref[...]
    lane = lax.broadcasted_iota(jnp.int32, sb.shape, 1)
    i0 = jnp.argmax(sb, axis=1)[:, None]
    sb2 = jnp.where(lane == i0, -jnp.inf, sb)
    i1 = jnp.argmax(sb2, axis=1)[:, None]
    s0 = jnp.sum(jnp.where(lane == i0, s, 0.0), axis=1, keepdims=True)
    s1 = jnp.sum(jnp.where(lane == i1, s, 0.0), axis=1, keepdims=True)
    eidx_ref[...] = jnp.concatenate([i0, i1], axis=1)
    g01 = jnp.concatenate(
        [jax.nn.sigmoid(s0 - s1), jax.nn.sigmoid(s1 - s0)], axis=1)
    bt = x.shape[0]
    gates_ref[...] = jnp.broadcast_to(g01[:, :, None],
                                      (bt, 2, 16)).reshape(2 * bt, 16)
    oh = (jnp.where(lane == i0, 1, 0) + jnp.where(lane == i1, 1, 0))
    ch = bt // 256
    csum = jnp.sum(oh.reshape(ch, 256, N), axis=1)
    cnts_ref[...] = jnp.concatenate(
        [csum, jnp.zeros((ch, 16 - N), jnp.int32)], axis=1)


def _router(x_MD, Wr, gate_b, sel_b):
    BTR = 2048
    nt = M // BTR
    return pl.pallas_call(
        _router_body,
        grid=(nt,),
        in_specs=[
            pl.BlockSpec((BTR, D), lambda t: (t, 0)),
            pl.BlockSpec((D, N), lambda t: (0, 0)),
            pl.BlockSpec((1, N), lambda t: (0, 0)),
            pl.BlockSpec((1, N), lambda t: (0, 0)),
        ],
        out_specs=[
            pl.BlockSpec((BTR, K), lambda t: (t, 0)),
            pl.BlockSpec((K * BTR, 16), lambda t: (t, 0)),
            pl.BlockSpec((BTR // 256, 16), lambda t: (t, 0)),
        ],
        out_shape=[
            jax.ShapeDtypeStruct((M, K), jnp.int32),
            jax.ShapeDtypeStruct((P, 16), jnp.float32),
            jax.ShapeDtypeStruct((NW, 16), jnp.int32),
        ],
    )(x_MD, Wr, gate_b, sel_b)


def _dispatch_body(pairs_hbm, cnts_hbm, x_hbm,
                   slots_hbm, xs_hbm, te_hbm,
                   e_v, cnt_v, slot_v, tok_v, te_v, rows_v,
                   sem):
    nc = 2
    wid = lax.axis_index("s") * nc + lax.axis_index("c")
    base = wid * CH
    pltpu.sync_copy(pairs_hbm.at[pl.ds(base, CH)], e_v)
    pltpu.sync_copy(cnts_hbm, cnt_v)

    lane = lax.iota(jnp.int32, 16)
    total = jnp.zeros((16,), jnp.int32)
    basev = jnp.zeros((16,), jnp.int32)
    for r in range(NW):
        row = cnt_v[r]
        total = total + row
        basev = basev + jnp.where(wid > r, row, 0)
    padded = ((total + (BT - 1)) >> 9) << 9
    csum = plsc.cumsum(padded)
    offs = csum - padded
    basev = offs + basev

    for c in range(3):
        jv = (lane + c * 16) * BT
        acc = jnp.zeros((16,), jnp.int32)
        for e in range(N - 1):
            ce = jnp.sum(jnp.where(lane == e, csum, 0))
            acc = acc + jnp.where(jv >= ce, 1, 0)
        te_v[pl.ds(c * 16, 16)] = acc

    @pl.when(wid == 0)
    def _write_te():
        pltpu.sync_copy(te_v, te_hbm)

    def vbody(i, basev):
        v = e_v[pl.ds(i * 16, 16)]
        slot = jnp.zeros((16,), jnp.int32)
        for e in range(N):
            m = v == e
            mi = jnp.where(m, 1, 0)
            rank = plsc.cumsum(mi)
            be = jnp.sum(jnp.where(lane == e, basev, 0))
            slot = jnp.where(m, be + rank - 1, slot)
            basev = basev + jnp.where(lane == e, jnp.sum(mi), 0)
        slot_v[i // 4, pl.ds((i % 4) * 16, 16)] = slot
        tok_v[i // 4, pl.ds((i % 4) * 16, 16)] = (base + i * 16 + lane) >> 1
        return basev

    basev = lax.fori_loop(0, CH // 16, vbody, basev)

    pltpu.sync_copy(slot_v, slots_hbm.at[wid])

    for c in range(NSUB):
        pltpu.async_copy(x_hbm.at[tok_v.at[c]], rows_v, sem).wait()
        pltpu.async_copy(rows_v, xs_hbm.at[slot_v.at[c]], sem).wait()


def _dispatch(pairs, cnts, x_MD):
    mesh = plsc.VectorSubcoreMesh(core_axis_name="c", subcore_axis_name="s")
    f = pl.kernel(
        _dispatch_body,
        mesh=mesh,
        compiler_params=pltpu.CompilerParams(needs_layout_passes=False),
        out_type=[
            jax.ShapeDtypeStruct((NW, NSUB, SUB), jnp.int32),
            jax.ShapeDtypeStruct((P_PAD, D), jnp.float32),
            jax.ShapeDtypeStruct((48,), jnp.int32),
        ],
        scratch_types=[
            pltpu.VMEM((CH,), jnp.int32),
            pltpu.VMEM((NW, 16), jnp.int32),
            pltpu.VMEM((NSUB, SUB), jnp.int32),
            pltpu.VMEM((NSUB, SUB), jnp.int32),
            pltpu.VMEM((48,), jnp.int32),
            pltpu.VMEM((SUB, D), jnp.float32),
            pltpu.SemaphoreType.DMA,
        ],
    )
    return f(pairs, cnts, x_MD)


def _gmm_body(te_ref, x_ref, w1_ref, w2_ref, out_ref):
    xb = x_ref[...].astype(jnp.bfloat16)
    h = lax.dot_general(xb, w1_ref[0], (((1,), (0,)), ((), ())),
                        preferred_element_type=jnp.float32)
    a = h[:, :H]
    b = h[:, H:]
    g = (a * jax.nn.sigmoid(a) * b).astype(jnp.bfloat16)
    o = lax.dot_general(g, w2_ref[0], (((1,), (0,)), ((), ())),
                        preferred_element_type=jnp.float32)
    out_ref[...] = o


def _gmm(te, x_sorted, W1b, W2b):
    spec = pltpu.PrefetchScalarGridSpec(
        num_scalar_prefetch=1,
        grid=(NT,),
        in_specs=[
            pl.BlockSpec((BT, D), lambda j, te_ref: (j, 0)),
            pl.BlockSpec((1, D, H2), lambda j, te_ref: (te_ref[j], 0, 0)),
            pl.BlockSpec((1, H, D), lambda j, te_ref: (te_ref[j], 0, 0)),
        ],
        out_specs=pl.BlockSpec((BT, D), lambda j, te_ref: (j, 0)),
    )
    return pl.pallas_call(
        _gmm_body,
        grid_spec=spec,
        out_shape=jax.ShapeDtypeStruct((P_PAD, D), jnp.float32),
    )(te, x_sorted, W1b, W2b)


def _combine_body(slots_hbm, gates_hbm, os_hbm, y_hbm,
                  slot_v, gb_v, rows_v, sem):
    nc = 2
    wid = lax.axis_index("s") * nc + lax.axis_index("c")
    pltpu.sync_copy(slots_hbm.at[wid], slot_v)
    pltpu.sync_copy(gates_hbm.at[pl.ds(wid * CH, CH)], gb_v)
    tok_base = wid * (CH // K)
    for c in range(NSUB):
        pltpu.async_copy(os_hbm.at[slot_v.at[c]], rows_v, sem).wait()

        def ibody(i, carry):
            g0 = gb_v[c * SUB + 2 * i]
            g1 = gb_v[c * SUB + 2 * i + 1]

            def hbody(h, carry2):
                sl = pl.ds(h * 16, 16)
                rows_v[i, sl] = (g0 * rows_v[2 * i, sl]
                                 + g1 * rows_v[2 * i + 1, sl])
                return carry2

            lax.fori_loop(0, D // 16, hbody, 0)
            return carry

        lax.fori_loop(0, SUB // K, ibody, 0)
        pltpu.sync_copy(rows_v.at[pl.ds(0, SUB // K)],
                        y_hbm.at[pl.ds(tok_base + c * (SUB // K), SUB // K)])


def _combine(slots, gates_b, out_sorted):
    mesh = plsc.VectorSubcoreMesh(core_axis_name="c", subcore_axis_name="s")
    f = pl.kernel(
        _combine_body,
        mesh=mesh,
        compiler_params=pltpu.CompilerParams(needs_layout_passes=False),
        out_type=jax.ShapeDtypeStruct((M, D), jnp.float32),
        scratch_types=[
            pltpu.VMEM((NSUB, SUB), jnp.int32),
            pltpu.VMEM((CH, 16), jnp.float32),
            pltpu.VMEM((SUB, D), jnp.float32),
            pltpu.SemaphoreType.DMA,
        ],
    )
    return f(slots, gates_b, out_sorted)


def kernel(x_BSD, Wr, br, W1, b1, W2, b2, biases_N):
    B, S, Dd = x_BSD.shape
    assert (B * S, Dd, Wr.shape[1], W2.shape[1]) == (M, D, N, H)
    x_MD = x_BSD.reshape(M, D)
    gate_b = br.reshape(1, N)
    sel_b = biases_N.reshape(1, N)
    W1b = W1.astype(jnp.bfloat16)
    W2b = W2.astype(jnp.bfloat16)

    eidx, gates_b, cnts = _router(x_MD, Wr, gate_b, sel_b)
    pairs = eidx.reshape(P)
    slots, x_sorted, te = _dispatch(pairs, cnts, x_MD)
    out_sorted = _gmm(te, x_sorted, W1b, W2b)
    y = _combine(slots, gates_b, out_sorted)
    return y.reshape(B, S, Dd)


# speedup vs baseline: 2.3461x; 1.0704x over previous
import functools

import jax
import jax.numpy as jnp
from jax import lax
from jax.experimental import pallas as pl
from jax.experimental.pallas import tpu as pltpu
from jax.experimental.pallas import tpu_sc as plsc

M = 8192
D = 768
H2 = 1536
H = 768
N = 8
K = 2
P = M * K
BT = 512
NW = 32
CH = P // NW
P_PAD = P + N * BT
NT = P_PAD // BT
SUB = 32
NSUB = CH // SUB


def _router_body(x_ref, wr_ref, gate_b_ref, sel_b_ref,
                 eidx_ref, gates_ref, cnts_ref):
    x = x_ref[...]
    s = lax.dot_general(x, wr_ref[...], (((1,), (0,)), ((), ())),
                        preferred_element_type=jnp.float32)
    s = s + gate_b_ref[...]
    sb = s + sel_b_ref[...]
    lane = lax.broadcasted_iota(jnp.int32, sb.shape, 1)
    i0 = jnp.argmax(sb, axis=1)[:, None]
    sb2 = jnp.where(lane == i0, -jnp.inf, sb)
    i1 = jnp.argmax(sb2, axis=1)[:, None]
    s0 = jnp.sum(jnp.where(lane == i0, s, 0.0), axis=1, keepdims=True)
    s1 = jnp.sum(jnp.where(lane == i1, s, 0.0), axis=1, keepdims=True)
    eidx_ref[...] = jnp.concatenate([i0, i1], axis=1)
    g01 = jnp.concatenate(
        [jax.nn.sigmoid(s0 - s1), jax.nn.sigmoid(s1 - s0)], axis=1)
    bt = x.shape[0]
    gates_ref[...] = jnp.broadcast_to(g01[:, :, None],
                                      (bt, 2, 16)).reshape(2 * bt, 16)
    oh = (jnp.where(lane == i0, 1, 0) + jnp.where(lane == i1, 1, 0))
    ch = bt // 256
    csum = jnp.sum(oh.reshape(ch, 256, N), axis=1)
    cnts_ref[...] = jnp.concatenate(
        [csum, jnp.zeros((ch, 16 - N), jnp.int32)], axis=1)


def _router(x_MD, Wr, gate_b, sel_b):
    BTR = 2048
    nt = M // BTR
    return pl.pallas_call(
        _router_body,
        grid=(nt,),
        in_specs=[
            pl.BlockSpec((BTR, D), lambda t: (t, 0)),
            pl.BlockSpec((D, N), lambda t: (0, 0)),
            pl.BlockSpec((1, N), lambda t: (0, 0)),
            pl.BlockSpec((1, N), lambda t: (0, 0)),
        ],
        out_specs=[
            pl.BlockSpec((BTR, K), lambda t: (t, 0)),
            pl.BlockSpec((K * BTR, 16), lambda t: (t, 0)),
            pl.BlockSpec((BTR // 256, 16), lambda t: (t, 0)),
        ],
        out_shape=[
            jax.ShapeDtypeStruct((M, K), jnp.int32),
            jax.ShapeDtypeStruct((P, 16), jnp.float32),
            jax.ShapeDtypeStruct((NW, 16), jnp.int32),
        ],
    )(x_MD, Wr, gate_b, sel_b)


def _dispatch_body(pairs_hbm, cnts_hbm, x_hbm,
                   slots_hbm, xs_hbm, te_hbm,
                   e_v, cnt_v, slot_v, tok_v, te_v, rows_v,
                   gsem0, gsem1, ssem0, ssem1):
    nc = 2
    wid = lax.axis_index("s") * nc + lax.axis_index("c")
    base = wid * CH
    pltpu.sync_copy(pairs_hbm.at[pl.ds(base, CH)], e_v)
    pltpu.sync_copy(cnts_hbm, cnt_v)

    lane = lax.iota(jnp.int32, 16)
    total = jnp.zeros((16,), jnp.int32)
    basev = jnp.zeros((16,), jnp.int32)
    for r in range(NW):
        row = cnt_v[r]
        total = total + row
        basev = basev + jnp.where(wid > r, row, 0)
    padded = ((total + (BT - 1)) >> 9) << 9
    csum = plsc.cumsum(padded)
    offs = csum - padded
    basev = offs + basev

    for c in range(3):
        jv = (lane + c * 16) * BT
        acc = jnp.zeros((16,), jnp.int32)
        for e in range(N - 1):
            ce = jnp.sum(jnp.where(lane == e, csum, 0))
            acc = acc + jnp.where(jv >= ce, 1, 0)
        te_v[pl.ds(c * 16, 16)] = acc

    @pl.when(wid == 0)
    def _write_te():
        pltpu.sync_copy(te_v, te_hbm)

    def vbody(i, basev):
        v = e_v[pl.ds(i * 16, 16)]
        slot = jnp.zeros((16,), jnp.int32)
        for e in range(N):
            m = v == e
            mi = jnp.where(m, 1, 0)
            rank = plsc.cumsum(mi)
            be = jnp.sum(jnp.where(lane == e, basev, 0))
            slot = jnp.where(m, be + rank - 1, slot)
            basev = basev + jnp.where(lane == e, jnp.sum(mi), 0)
        vr = SUB // 16
        slot_v[i // vr, pl.ds((i % vr) * 16, 16)] = slot
        tok_v[i // vr, pl.ds((i % vr) * 16, 16)] = (base + i * 16 + lane) >> 1
        return basev

    basev = lax.fori_loop(0, CH // 16, vbody, basev)

    pltpu.sync_copy(slot_v, slots_hbm.at[wid])

    gh = [None, None]
    sh = [None, None]
    gh[0] = pltpu.async_copy(x_hbm.at[tok_v.at[0]], rows_v.at[0], gsem0)
    gsems = [gsem0, gsem1]
    ssems = [ssem0, ssem1]
    for c in range(NSUB):
        b = c & 1
        nb = (c + 1) & 1
        if c + 1 < NSUB:
            if c >= 1:
                sh[nb].wait()
            gh[nb] = pltpu.async_copy(x_hbm.at[tok_v.at[c + 1]],
                                      rows_v.at[nb], gsems[nb])
        gh[b].wait()
        sh[b] = pltpu.async_copy(rows_v.at[b], xs_hbm.at[slot_v.at[c]],
                                 ssems[b])
    sh[0].wait()
    sh[1].wait()


def _dispatch(pairs, cnts, x_MD):
    mesh = plsc.VectorSubcoreMesh(core_axis_name="c", subcore_axis_name="s")
    f = pl.kernel(
        _dispatch_body,
        mesh=mesh,
        compiler_params=pltpu.CompilerParams(needs_layout_passes=False),
        out_type=[
            jax.ShapeDtypeStruct((NW, NSUB, SUB), jnp.int32),
            jax.ShapeDtypeStruct((P_PAD, D), jnp.float32),
            jax.ShapeDtypeStruct((48,), jnp.int32),
        ],
        scratch_types=[
            pltpu.VMEM((CH,), jnp.int32),
            pltpu.VMEM((NW, 16), jnp.int32),
            pltpu.VMEM((NSUB, SUB), jnp.int32),
            pltpu.VMEM((NSUB, SUB), jnp.int32),
            pltpu.VMEM((48,), jnp.int32),
            pltpu.VMEM((2, SUB, D), jnp.float32),
            pltpu.SemaphoreType.DMA,
            pltpu.SemaphoreType.DMA,
            pltpu.SemaphoreType.DMA,
            pltpu.SemaphoreType.DMA,
        ],
    )
    return f(pairs, cnts, x_MD)


def _gmm_body(te_ref, x_ref, w1_ref, w2_ref, out_ref):
    xb = x_ref[...].astype(jnp.bfloat16)
    h = lax.dot_general(xb, w1_ref[0], (((1,), (0,)), ((), ())),
                        preferred_element_type=jnp.float32)
    a = h[:, :H]
    b = h[:, H:]
    g = (a * jax.nn.sigmoid(a) * b).astype(jnp.bfloat16)
    o = lax.dot_general(g, w2_ref[0], (((1,), (0,)), ((), ())),
                        preferred_element_type=jnp.float32)
    out_ref[...] = o


def _gmm(te, x_sorted, W1b, W2b):
    spec = pltpu.PrefetchScalarGridSpec(
        num_scalar_prefetch=1,
        grid=(NT,),
        in_specs=[
            pl.BlockSpec((BT, D), lambda j, te_ref: (j, 0)),
            pl.BlockSpec((1, D, H2), lambda j, te_ref: (te_ref[j], 0, 0)),
            pl.BlockSpec((1, H, D), lambda j, te_ref: (te_ref[j], 0, 0)),
        ],
        out_specs=pl.BlockSpec((BT, D), lambda j, te_ref: (j, 0)),
    )
    return pl.pallas_call(
        _gmm_body,
        grid_spec=spec,
        out_shape=jax.ShapeDtypeStruct((P_PAD, D), jnp.float32),
    )(te, x_sorted, W1b, W2b)


def _combine_body(slots_hbm, gates_hbm, os_hbm, y_hbm,
                  slot_v, gb_v, rows_v, gsem0, gsem1):
    nc = 2
    wid = lax.axis_index("s") * nc + lax.axis_index("c")
    pltpu.sync_copy(slots_hbm.at[wid], slot_v)
    pltpu.sync_copy(gates_hbm.at[pl.ds(wid * CH, CH)], gb_v)
    tok_base = wid * (CH // K)
    gsems = [gsem0, gsem1]
    gh = [None, None]
    gh[0] = pltpu.async_copy(os_hbm.at[slot_v.at[0]], rows_v.at[0], gsems[0])
    for c in range(NSUB):
        b = c & 1
        nb = (c + 1) & 1
        if c + 1 < NSUB:
            gh[nb] = pltpu.async_copy(os_hbm.at[slot_v.at[c + 1]],
                                      rows_v.at[nb], gsems[nb])
        gh[b].wait()

        def ibody(i, carry):
            g0 = gb_v[c * SUB + 2 * i]
            g1 = gb_v[c * SUB + 2 * i + 1]

            def hbody(h, carry2):
                sl = pl.ds(h * 16, 16)
                rows_v[b, i, sl] = (g0 * rows_v[b, 2 * i, sl]
                                    + g1 * rows_v[b, 2 * i + 1, sl])
                return carry2

            lax.fori_loop(0, D // 16, hbody, 0)
            return carry

        lax.fori_loop(0, SUB // K, ibody, 0)
        pltpu.sync_copy(rows_v.at[b, pl.ds(0, SUB // K)],
                        y_hbm.at[pl.ds(tok_base + c * (SUB // K), SUB // K)])


def _combine(slots, gates_b, out_sorted):
    mesh = plsc.VectorSubcoreMesh(core_axis_name="c", subcore_axis_name="s")
    f = pl.kernel(
        _combine_body,
        mesh=mesh,
        compiler_params=pltpu.CompilerParams(needs_layout_passes=False),
        out_type=jax.ShapeDtypeStruct((M, D), jnp.float32),
        scratch_types=[
            pltpu.VMEM((NSUB, SUB), jnp.int32),
            pltpu.VMEM((CH, 16), jnp.float32),
            pltpu.VMEM((2, SUB, D), jnp.float32),
            pltpu.SemaphoreType.DMA,
            pltpu.SemaphoreType.DMA,
        ],
    )
    return f(slots, gates_b, out_sorted)


def kernel(x_BSD, Wr, br, W1, b1, W2, b2, biases_N):
    B, S, Dd = x_BSD.shape
    assert (B * S, Dd, Wr.shape[1], W2.shape[1]) == (M, D, N, H)
    x_MD = x_BSD.reshape(M, D)
    gate_b = br.reshape(1, N)
    sel_b = biases_N.reshape(1, N)
    W1b = W1.astype(jnp.bfloat16)
    W2b = W2.astype(jnp.bfloat16)

    eidx, gates_b, cnts = _router(x_MD, Wr, gate_b, sel_b)
    pairs = eidx.reshape(P)
    slots, x_sorted, te = _dispatch(pairs, cnts, x_MD)
    out_sorted = _gmm(te, x_sorted, W1b, W2b)
    y = _combine(slots, gates_b, out_sorted)
    return y.reshape(B, S, Dd)
